# baseline (speedup 1.0000x reference)
import functools

import jax
import jax.numpy as jnp
from jax.experimental import pallas as pl
from jax.experimental.pallas import tpu as pltpu

N = 8192
BQ = 128
BK = 256
NKB = N // BK
NQB = N // BQ
NSLICES = 8


def _nn_kernel(q_ref, k_ref, acc_ref, min_ref, idx_ref):
    s = pl.program_id(0)
    qb = pl.program_id(1)

    qx = q_ref[0, 0:1, :]
    qy = q_ref[0, 1:2, :]
    qz = q_ref[0, 2:3, :]

    iota_s = jax.lax.broadcasted_iota(jnp.int32, (BK, BQ), 0)
    big = jnp.int32(2 * BK)

    def body(kb, carry):
        rmin, ridx = carry
        base = kb * BK
        kx = k_ref[0, pl.ds(base, BK), 0:1]
        ky = k_ref[0, pl.ds(base, BK), 1:2]
        kz = k_ref[0, pl.ds(base, BK), 2:3]
        dx = kx - qx
        dy = ky - qy
        dz = kz - qz
        d = dx * dx + dy * dy + dz * dz
        bm = jnp.min(d, axis=0, keepdims=True)
        barg = jnp.min(
            jnp.where(d == bm, iota_s, big), axis=0, keepdims=True
        )
        upd = bm < rmin
        ridx = jnp.where(upd, barg + base, ridx)
        rmin = jnp.where(upd, bm, rmin)
        return rmin, ridx

    rmin0 = jnp.full((1, BQ), jnp.inf, jnp.float32)
    ridx0 = jnp.zeros((1, BQ), jnp.int32)
    rmin, ridx = jax.lax.fori_loop(0, NKB, body, (rmin0, ridx0))

    min_ref[0, 0, :] = rmin[0]
    idx_ref[0, 0, pl.ds(qb * BQ, BQ)] = ridx[0]

    @pl.when(jnp.logical_and(s == 0, qb == 0))
    def _init():
        acc_ref[0, 0] = 0.0

    acc_ref[0, 0] += jnp.sum(rmin)


@functools.partial(jax.jit)
def _run(q, k):
    acc, dmin, idx = pl.pallas_call(
        _nn_kernel,
        grid=(NSLICES, NQB),
        in_specs=[
            pl.BlockSpec((1, 3, BQ), lambda s, qb: (s, 0, qb)),
            pl.BlockSpec((1, N, 3), lambda s, qb: (s, 0, 0)),
        ],
        out_specs=[
            pl.BlockSpec((1, 1), lambda s, qb: (0, 0), memory_space=pltpu.SMEM),
            pl.BlockSpec((1, 1, BQ), lambda s, qb: (s, 0, qb)),
            pl.BlockSpec((1, 1, N), lambda s, qb: (s, 0, 0)),
        ],
        out_shape=[
            jax.ShapeDtypeStruct((1, 1), jnp.float32),
            jax.ShapeDtypeStruct((NSLICES, 1, N), jnp.float32),
            jax.ShapeDtypeStruct((NSLICES, 1, N), jnp.int32),
        ],
    )(q, k)
    return acc, dmin, idx


def kernel(pred_points, true_points):
    q = jnp.concatenate([pred_points, true_points], axis=0).transpose(0, 2, 1)
    k = jnp.concatenate([true_points, pred_points], axis=0)
    acc, _, idx = _run(q, k)
    idx = idx.reshape(NSLICES, N)
    loss = acc[0, 0] / jnp.float32(4 * N)
    return loss, idx[:4], idx[4:]


# speedup vs baseline: 2.3377x; 2.3377x over previous
import functools

import jax
import jax.numpy as jnp
from jax.experimental import pallas as pl
from jax.experimental.pallas import tpu as pltpu

N = 8192
BQ = 64
LK = 128
NKB = N // LK
NQB = N // BQ
NSLICES = 8
BIGI = 1 << 30


def _nn_kernel(q_ref, k_ref, acc_ref, idx_ref):
    s = pl.program_id(0)
    qb = pl.program_id(1)

    qxb = jnp.broadcast_to(q_ref[0, :, 0:1], (BQ, LK))
    qyb = jnp.broadcast_to(q_ref[0, :, 1:2], (BQ, LK))
    qzb = jnp.broadcast_to(q_ref[0, :, 2:3], (BQ, LK))

    def body(kb, carry):
        runvals, runkb = carry
        base = kb * LK
        kx = k_ref[0, 0:1, pl.ds(base, LK)]
        ky = k_ref[0, 1:2, pl.ds(base, LK)]
        kz = k_ref[0, 2:3, pl.ds(base, LK)]
        dx = qxb - kx
        dy = qyb - ky
        dz = qzb - kz
        d = dx * dx + dy * dy + dz * dz
        m = d < runvals
        runvals = jnp.where(m, d, runvals)
        runkb = jnp.where(m, kb, runkb)
        return runvals, runkb

    rv0 = jnp.full((BQ, LK), jnp.inf, jnp.float32)
    rk0 = jnp.zeros((BQ, LK), jnp.int32)
    runvals, runkb = jax.lax.fori_loop(0, NKB, body, (rv0, rk0))

    lane = jax.lax.broadcasted_iota(jnp.int32, (BQ, LK), 1)
    runkey = runkb * LK + lane
    bm = jnp.min(runvals, axis=1, keepdims=True)
    ridx = jnp.min(
        jnp.where(runvals == bm, runkey, BIGI), axis=1, keepdims=True
    )
    idx_ref[0, :, :] = ridx

    @pl.when(jnp.logical_and(s == 0, qb == 0))
    def _init():
        acc_ref[0, 0] = 0.0

    acc_ref[0, 0] += jnp.sum(bm)


@functools.partial(jax.jit)
def _run(q, k):
    acc, idx = pl.pallas_call(
        _nn_kernel,
        grid=(NSLICES, NQB),
        in_specs=[
            pl.BlockSpec((1, BQ, 3), lambda s, qb: (s, qb, 0)),
            pl.BlockSpec((1, 3, N), lambda s, qb: (s, 0, 0)),
        ],
        out_specs=[
            pl.BlockSpec((1, 1), lambda s, qb: (0, 0), memory_space=pltpu.SMEM),
            pl.BlockSpec((1, BQ, 1), lambda s, qb: (s, qb, 0)),
        ],
        out_shape=[
            jax.ShapeDtypeStruct((1, 1), jnp.float32),
            jax.ShapeDtypeStruct((NSLICES, N, 1), jnp.int32),
        ],
    )(q, k)
    return acc, idx


def kernel(pred_points, true_points):
    q = jnp.concatenate([pred_points, true_points], axis=0)
    k = jnp.concatenate([true_points, pred_points], axis=0).transpose(0, 2, 1)
    acc, idx = _run(q, k)
    idx = idx.reshape(NSLICES, N)
    loss = acc[0, 0] / jnp.float32(4 * N)
    return loss, idx[:4], idx[4:]


# speedup vs baseline: 2.8167x; 1.2049x over previous
import functools

import jax
import jax.numpy as jnp
from jax.experimental import pallas as pl
from jax.experimental.pallas import tpu as pltpu

N = 8192
BQ = 64
LK = 128
NKB = N // LK
NQB = N // BQ
NSLICES = 8
BIGI = 1 << 30


def _nn_kernel(q_ref, k_ref, acc_ref, idx_ref):
    s = pl.program_id(0)
    qb = pl.program_id(1)

    qxb = jnp.broadcast_to(q_ref[0, :, 0:1], (BQ, LK))
    qyb = jnp.broadcast_to(q_ref[0, :, 1:2], (BQ, LK))
    qzb = jnp.broadcast_to(q_ref[0, :, 2:3], (BQ, LK))

    def body(kb, carry):
        runvals, runkb = carry
        base = kb * LK
        kx = k_ref[0, 0:1, pl.ds(base, LK)]
        ky = k_ref[0, 1:2, pl.ds(base, LK)]
        kz = k_ref[0, 2:3, pl.ds(base, LK)]
        dx = qxb - kx
        dy = qyb - ky
        dz = qzb - kz
        d = dx * dx + dy * dy + dz * dz
        m = d < runvals
        runvals = jnp.where(m, d, runvals)
        runkb = jnp.where(m, kb, runkb)
        return runvals, runkb

    rv0 = jnp.full((BQ, LK), jnp.inf, jnp.float32)
    rk0 = jnp.zeros((BQ, LK), jnp.int32)
    runvals, runkb = jax.lax.fori_loop(0, NKB, body, (rv0, rk0), unroll=8)

    lane = jax.lax.broadcasted_iota(jnp.int32, (BQ, LK), 1)
    runkey = runkb * LK + lane
    bm = jnp.min(runvals, axis=1, keepdims=True)
    ridx = jnp.min(
        jnp.where(runvals == bm, runkey, BIGI), axis=1, keepdims=True
    )
    idx_ref[0, :, :] = ridx

    @pl.when(jnp.logical_and(s == 0, qb == 0))
    def _init():
        acc_ref[0, 0] = 0.0

    acc_ref[0, 0] += jnp.sum(bm)


@functools.partial(jax.jit)
def _run(q, k):
    acc, idx = pl.pallas_call(
        _nn_kernel,
        grid=(NSLICES, NQB),
        in_specs=[
            pl.BlockSpec((1, BQ, 3), lambda s, qb: (s, qb, 0)),
            pl.BlockSpec((1, 3, N), lambda s, qb: (s, 0, 0)),
        ],
        out_specs=[
            pl.BlockSpec((1, 1), lambda s, qb: (0, 0), memory_space=pltpu.SMEM),
            pl.BlockSpec((1, BQ, 1), lambda s, qb: (s, qb, 0)),
        ],
        out_shape=[
            jax.ShapeDtypeStruct((1, 1), jnp.float32),
            jax.ShapeDtypeStruct((NSLICES, N, 1), jnp.int32),
        ],
    )(q, k)
    return acc, idx


def kernel(pred_points, true_points):
    q = jnp.concatenate([pred_points, true_points], axis=0)
    k = jnp.concatenate([true_points, pred_points], axis=0).transpose(0, 2, 1)
    acc, idx = _run(q, k)
    idx = idx.reshape(NSLICES, N)
    loss = acc[0, 0] / jnp.float32(4 * N)
    return loss, idx[:4], idx[4:]
